# Initial kernel scaffold; baseline (speedup 1.0000x reference)
import jax
import jax.numpy as jnp
from jax.experimental import pallas as pl


def kernel(x, adj_indices, adj_values, W, b):
    raise NotImplementedError("write your pallas kernel here")


# speedup vs baseline: 4.7314x; 4.7314x over previous
import functools

import jax
import jax.numpy as jnp
from jax import lax
from jax.experimental import pallas as pl
from jax.experimental.pallas import tpu as pltpu
from jax.experimental.pallas import tpu_sc as plsc

N_NODES = 10000
N_EDGES = 320000
DIM = 128

NC = 2
NS = 16
NW = NC * NS
CH = 128
NCHUNKS = N_EDGES // CH
BASE_CHUNKS = NCHUNKS // NW
EXTRA = NCHUNKS - BASE_CHUNKS * NW
ROWS_PER_TILE = 624
TAIL_ROWS = N_NODES - NS * ROWS_PER_TILE


def _mm_body(x_ref, w_ref, b_ref, o_ref):
    o_ref[...] = lax.dot_general(
        x_ref[...], w_ref[...], (((1,), (1,)), ((), ())),
        preferred_element_type=jnp.float32) + b_ref[...]


_matmul = pl.pallas_call(
    _mm_body,
    grid=(10,),
    in_specs=[
        pl.BlockSpec((1000, DIM), lambda i: (i, 0)),
        pl.BlockSpec((DIM, DIM), lambda i: (0, 0)),
        pl.BlockSpec((1, DIM), lambda i: (0, 0)),
    ],
    out_specs=pl.BlockSpec((1000, DIM), lambda i: (i, 0)),
    out_shape=jax.ShapeDtypeStruct((N_NODES, DIM), jnp.float32),
)


def _add_body(p_ref, o_ref):
    o_ref[...] = p_ref[0] + p_ref[1]


_combine = pl.pallas_call(
    _add_body,
    grid=(10,),
    in_specs=[pl.BlockSpec((2, 1000, DIM), lambda i: (0, i, 0))],
    out_specs=pl.BlockSpec((1000, DIM), lambda i: (i, 0)),
    out_shape=jax.ShapeDtypeStruct((N_NODES, DIM), jnp.float32),
)


_MESH = plsc.VectorSubcoreMesh(
    core_axis_name="c", subcore_axis_name="s", num_cores=NC, num_subcores=NS)


@functools.partial(
    pl.kernel,
    out_type=jax.ShapeDtypeStruct((NC, N_NODES, DIM), jnp.float32),
    mesh=_MESH,
    compiler_params=pltpu.CompilerParams(needs_layout_passes=False),
    scratch_types=[
        pltpu.VMEM((CH,), jnp.int32),
        pltpu.VMEM((CH,), jnp.int32),
        pltpu.VMEM((CH,), jnp.float32),
        pltpu.VMEM((CH, DIM), jnp.float32),
        pltpu.VMEM_SHARED((N_NODES, DIM), jnp.float32),
        pltpu.SemaphoreType.DMA,
    ],
)
def _spmm(h_hbm, row_hbm, col_hbm, vals_hbm, out_hbm,
          col_v, row_v, vals_v, rows_v, acc, gsem):
    cid = lax.axis_index("c")
    sid = lax.axis_index("s")
    wid = sid * NC + cid

    zv = jnp.zeros((16,), jnp.float32)

    def _zero_body(e, carry):
        for f in range(DIM // 16):
            rows_v[e, pl.ds(f * 16, 16)] = zv
        return carry

    lax.fori_loop(0, CH, _zero_body, 0)
    r0 = sid * ROWS_PER_TILE
    for j in range(4):
        pltpu.sync_copy(rows_v, acc.at[pl.ds(r0 + j * CH, CH)])
    pltpu.sync_copy(rows_v.at[pl.ds(0, ROWS_PER_TILE - 4 * CH)],
                    acc.at[pl.ds(r0 + 4 * CH, ROWS_PER_TILE - 4 * CH)])

    @pl.when(sid == NS - 1)
    def _zero_tail():
        pltpu.sync_copy(rows_v.at[pl.ds(0, TAIL_ROWS)],
                        acc.at[pl.ds(NS * ROWS_PER_TILE, TAIL_ROWS)])

    plsc.subcore_barrier()

    def _scale_body(e, carry):
        valv = plsc.load_gather(vals_v, [jnp.full((16,), e, jnp.int32)])
        for f in range(DIM // 16):
            sl = pl.ds(f * 16, 16)
            rows_v[e, sl] = rows_v[e, sl] * valv
        return carry

    def _chunk_body(i, carry):
        base = (wid + i * NW) * CH
        pltpu.sync_copy(col_hbm.at[pl.ds(base, CH)], col_v)
        pltpu.sync_copy(row_hbm.at[pl.ds(base, CH)], row_v)
        pltpu.sync_copy(vals_hbm.at[pl.ds(base, CH)], vals_v)
        pltpu.async_copy(h_hbm.at[col_v], rows_v, gsem).wait()
        lax.fori_loop(0, CH, _scale_body, 0)
        pltpu.sync_copy(rows_v, acc.at[row_v], add=True)
        return carry

    nch = jnp.where(wid < EXTRA, BASE_CHUNKS + 1, BASE_CHUNKS)
    lax.fori_loop(0, nch, _chunk_body, 0)

    plsc.subcore_barrier()
    pltpu.sync_copy(acc.at[pl.ds(r0, ROWS_PER_TILE)],
                    out_hbm.at[cid, pl.ds(r0, ROWS_PER_TILE)])

    @pl.when(sid == NS - 1)
    def _write_tail():
        pltpu.sync_copy(acc.at[pl.ds(NS * ROWS_PER_TILE, TAIL_ROWS)],
                        out_hbm.at[cid, pl.ds(NS * ROWS_PER_TILE, TAIL_ROWS)])


def kernel(x, adj_indices, adj_values, W, b):
    idx = adj_indices.astype(jnp.int32)
    row = idx[0]
    col = idx[1]
    h = _matmul(x, W, b.reshape(1, DIM))
    parts = _spmm(h, row, col, adj_values)
    return _combine(parts)
